# baseline (speedup 1.0000x reference)
import jax
import jax.numpy as jnp
from jax.experimental import pallas as pl
from jax.experimental.pallas import tpu as pltpu

B, T, N, FI, FO, H = 2, 30, 1000, 8, 8, 128
NEG = -1e30


TC = 5


def _gat_kernel(x_ref, adj_ref, wg_ref, asrc_ref, adst_ref, bg_ref, out_ref):
    for ci in range(TC):
        xb = x_ref[0, ci]
        h = jnp.dot(xb, wg_ref[...], preferred_element_type=jnp.float32)
        e_src = jax.lax.dot_general(
            h, asrc_ref[...], (((1,), (1,)), ((), ())),
            preferred_element_type=jnp.float32)
        e_dst = jax.lax.dot_general(
            adst_ref[...], h, (((1,), (1,)), ((), ())),
            preferred_element_type=jnp.float32)
        ms = jnp.max(e_src)
        md = jnp.max(e_dst)
        a_col = jnp.exp(e_src - ms)
        c_col = jnp.exp(0.2 * e_src - ms)
        b_row = jnp.exp(e_dst - md)
        d_row = jnp.exp(0.2 * e_dst - md)
        adjb = adj_ref[0, ci]
        jj = jax.lax.broadcasted_iota(jnp.int32, (N, N), 0)
        ii = jax.lax.broadcasted_iota(jnp.int32, (N, N), 1)
        mask = jnp.logical_or(adjb != 0.0, jj == ii)
        p0 = jnp.where(e_src >= -e_dst, a_col * b_row, c_col * d_row)
        p = jnp.where(mask, p0, 0.0)
        ones = jnp.ones((N, 1), dtype=jnp.float32)
        h_aug = jnp.concatenate([h, ones], axis=1)
        out_aug = jax.lax.dot_general(
            p, h_aug, (((0,), (0,)), ((), ())),
            preferred_element_type=jnp.float32)
        out_ref[ci, 0] = (out_aug[:, :FO] / out_aug[:, FO:FO + 1]
                          + bg_ref[...]).astype(jnp.bfloat16)


KD = 8
GCH = 4 * H // KD


def _lstm_kernel(seq_ref, wih_ref, whh_ref, bias_ref, wout_ref, bout_ref,
                 out_ref, acc_ref):
    k = pl.program_id(0)
    part = jax.lax.dot_general(
        seq_ref[...], wih_ref[...].astype(jnp.bfloat16),
        (((1,), (1,)), ((), ())),
        preferred_element_type=jnp.float32)
    for i in range(KD):
        @pl.when(k == i)
        def _(i=i):
            acc_ref[:, GCH * i:GCH * (i + 1)] = part

    @pl.when(k == KD - 1)
    def _():
        _lstm_tail(acc_ref, whh_ref, bias_ref, wout_ref, bout_ref, out_ref)


def _lstm_tail(acc_ref, whh_ref, bias_ref, wout_ref, bout_ref, out_ref):
    gates_x = acc_ref[...] + bias_ref[...]
    lane = jax.lax.broadcasted_iota(jnp.int32, (1, 4 * H), 1)
    sc = jnp.where(jnp.logical_and(lane >= 2 * H, lane < 3 * H), 1.0, 0.5)
    h = jnp.zeros((B, H), dtype=jnp.float32)
    c = jnp.zeros((B, H), dtype=jnp.float32)
    for t in range(T):
        gx = gates_x[2 * t:2 * t + 2]
        gates = gx + jax.lax.dot_general(
            h, whh_ref[...], (((1,), (1,)), ((), ())),
            preferred_element_type=jnp.float32)
        tg = jnp.tanh(gates * sc)
        i = tg[:, 0:H] + 1.0
        f = tg[:, H:2 * H] + 1.0
        g = tg[:, 2 * H:3 * H]
        o = tg[:, 3 * H:4 * H] + 1.0
        c = 0.5 * f * c + 0.5 * i * g
        h = 0.5 * o * jnp.tanh(c)
    pred = jnp.dot(h, wout_ref[...], preferred_element_type=jnp.float32)
    out_ref[...] = pred + bout_ref[...]


@jax.jit
def kernel(x, adj, W_gat, att_src, att_dst, b_gat, W_ih, W_hh, b_ih, b_hh,
           W_out, b_out):
    gat_out = pl.pallas_call(
        _gat_kernel,
        grid=(B, T // TC),
        in_specs=[
            pl.BlockSpec((1, TC, N, FI), lambda b, t: (b, t, 0, 0)),
            pl.BlockSpec((1, TC, N, N), lambda b, t: (b, t, 0, 0)),
            pl.BlockSpec((FI, FO), lambda b, t: (0, 0)),
            pl.BlockSpec((1, FO), lambda b, t: (0, 0)),
            pl.BlockSpec((1, FO), lambda b, t: (0, 0)),
            pl.BlockSpec((1, FO), lambda b, t: (0, 0)),
        ],
        out_specs=pl.BlockSpec((TC, 1, N, FO), lambda b, t: (t, b, 0, 0)),
        out_shape=jax.ShapeDtypeStruct((T, B, N, FO), jnp.bfloat16),
    )(x, adj, W_gat, att_src.reshape(1, FO), att_dst.reshape(1, FO),
      b_gat.reshape(1, FO))

    seq = gat_out.reshape(T * B, N * FO)
    bias = (b_ih + b_hh).reshape(1, 4 * H)

    pred = pl.pallas_call(
        _lstm_kernel,
        grid=(KD,),
        in_specs=[
            pl.BlockSpec((T * B, N * FO), lambda k: (0, 0)),
            pl.BlockSpec((GCH, N * FO), lambda k: (k, 0)),
            pl.BlockSpec((4 * H, H), lambda k: (0, 0)),
            pl.BlockSpec((1, 4 * H), lambda k: (0, 0)),
            pl.BlockSpec((H, N), lambda k: (0, 0)),
            pl.BlockSpec((1, N), lambda k: (0, 0)),
        ],
        out_specs=pl.BlockSpec((B, N), lambda k: (0, 0)),
        out_shape=jax.ShapeDtypeStruct((B, N), jnp.float32),
        scratch_shapes=[pltpu.VMEM((T * B, 4 * H), jnp.float32)],
    )(seq, W_ih, W_hh, bias, W_out, b_out.reshape(1, N))
    return pred


# speedup vs baseline: 1.0120x; 1.0120x over previous
import jax
import jax.numpy as jnp
from jax.experimental import pallas as pl
from jax.experimental.pallas import tpu as pltpu

B, T, N, FI, FO, H = 2, 30, 1000, 8, 8, 128
NEG = -1e30


TC = 5


def _gat_kernel(x_ref, adj_ref, wg_ref, asrc_ref, adst_ref, bg_ref, out_ref):
    for ci in range(TC):
        xb = x_ref[0, ci]
        h = jnp.dot(xb, wg_ref[...], preferred_element_type=jnp.float32)
        e_src = jax.lax.dot_general(
            h, asrc_ref[...], (((1,), (1,)), ((), ())),
            preferred_element_type=jnp.float32)
        e_dst = jax.lax.dot_general(
            adst_ref[...], h, (((1,), (1,)), ((), ())),
            preferred_element_type=jnp.float32)
        ms = jnp.max(e_src)
        md = jnp.max(e_dst)
        a_col = jnp.exp(e_src - ms)
        c_col = jnp.exp(0.2 * e_src - ms)
        b_row = jnp.exp(e_dst - md)
        d_row = jnp.exp(0.2 * e_dst - md)
        adjb = adj_ref[0, ci]
        jj = jax.lax.broadcasted_iota(jnp.int32, (N, N), 0)
        ii = jax.lax.broadcasted_iota(jnp.int32, (N, N), 1)
        mask = jnp.logical_or(adjb != 0.0, jj == ii)
        p0 = jnp.where(e_src >= -e_dst, a_col * b_row, c_col * d_row)
        p = jnp.where(mask, p0, 0.0)
        ones = jnp.ones((N, 1), dtype=jnp.float32)
        h_aug = jnp.concatenate([h, ones], axis=1)
        out_aug = jax.lax.dot_general(
            p, h_aug, (((0,), (0,)), ((), ())),
            preferred_element_type=jnp.float32)
        out_ref[ci, 0] = (out_aug[:, :FO] / out_aug[:, FO:FO + 1]
                          + bg_ref[...]).astype(jnp.bfloat16)


KD = 4
GCH = 4 * H // KD


def _lstm_kernel(seq_ref, wih_ref, whh_ref, bias_ref, wout_ref, bout_ref,
                 out_ref, acc_ref):
    k = pl.program_id(0)
    part = jax.lax.dot_general(
        seq_ref[...], wih_ref[...].astype(jnp.bfloat16),
        (((1,), (1,)), ((), ())),
        preferred_element_type=jnp.float32)
    for i in range(KD):
        @pl.when(k == i)
        def _(i=i):
            acc_ref[:, GCH * i:GCH * (i + 1)] = part

    @pl.when(k == KD - 1)
    def _():
        _lstm_tail(acc_ref, whh_ref, bias_ref, wout_ref, bout_ref, out_ref)


def _lstm_tail(acc_ref, whh_ref, bias_ref, wout_ref, bout_ref, out_ref):
    gates_x = acc_ref[...] + bias_ref[...]
    lane = jax.lax.broadcasted_iota(jnp.int32, (1, 4 * H), 1)
    sc = jnp.where(jnp.logical_and(lane >= 2 * H, lane < 3 * H), 1.0, 0.5)
    h = jnp.zeros((B, H), dtype=jnp.float32)
    c = jnp.zeros((B, H), dtype=jnp.float32)
    for t in range(T):
        gx = gates_x[2 * t:2 * t + 2]
        gates = gx + jax.lax.dot_general(
            h, whh_ref[...], (((1,), (1,)), ((), ())),
            preferred_element_type=jnp.float32)
        tg = jnp.tanh(gates * sc)
        i = tg[:, 0:H] + 1.0
        f = tg[:, H:2 * H] + 1.0
        g = tg[:, 2 * H:3 * H]
        o = tg[:, 3 * H:4 * H] + 1.0
        c = 0.5 * f * c + 0.5 * i * g
        h = 0.5 * o * jnp.tanh(c)
    pred = jnp.dot(h, wout_ref[...], preferred_element_type=jnp.float32)
    out_ref[...] = pred + bout_ref[...]


@jax.jit
def kernel(x, adj, W_gat, att_src, att_dst, b_gat, W_ih, W_hh, b_ih, b_hh,
           W_out, b_out):
    gat_out = pl.pallas_call(
        _gat_kernel,
        grid=(B, T // TC),
        in_specs=[
            pl.BlockSpec((1, TC, N, FI), lambda b, t: (b, t, 0, 0)),
            pl.BlockSpec((1, TC, N, N), lambda b, t: (b, t, 0, 0)),
            pl.BlockSpec((FI, FO), lambda b, t: (0, 0)),
            pl.BlockSpec((1, FO), lambda b, t: (0, 0)),
            pl.BlockSpec((1, FO), lambda b, t: (0, 0)),
            pl.BlockSpec((1, FO), lambda b, t: (0, 0)),
        ],
        out_specs=pl.BlockSpec((TC, 1, N, FO), lambda b, t: (t, b, 0, 0)),
        out_shape=jax.ShapeDtypeStruct((T, B, N, FO), jnp.bfloat16),
    )(x, adj, W_gat, att_src.reshape(1, FO), att_dst.reshape(1, FO),
      b_gat.reshape(1, FO))

    seq = gat_out.reshape(T * B, N * FO)
    bias = (b_ih + b_hh).reshape(1, 4 * H)

    pred = pl.pallas_call(
        _lstm_kernel,
        grid=(KD,),
        in_specs=[
            pl.BlockSpec((T * B, N * FO), lambda k: (0, 0)),
            pl.BlockSpec((GCH, N * FO), lambda k: (k, 0)),
            pl.BlockSpec((4 * H, H), lambda k: (0, 0)),
            pl.BlockSpec((1, 4 * H), lambda k: (0, 0)),
            pl.BlockSpec((H, N), lambda k: (0, 0)),
            pl.BlockSpec((1, N), lambda k: (0, 0)),
        ],
        out_specs=pl.BlockSpec((B, N), lambda k: (0, 0)),
        out_shape=jax.ShapeDtypeStruct((B, N), jnp.float32),
        scratch_shapes=[pltpu.VMEM((T * B, 4 * H), jnp.float32)],
    )(seq, W_ih, W_hh, bias, W_out, b_out.reshape(1, N))
    return pred
